# baseline (speedup 1.0000x reference)
import jax
import jax.numpy as jnp
from jax import lax
from jax.experimental import pallas as pl
from jax.experimental.pallas import tpu as pltpu
from jax.experimental.pallas import tpu_sc as plsc

VOCAB = 1000000
D = 32
BATCH = 16384
FIELDS = 26
B = BATCH * FIELDS

NC = 2
NS = 16
NW = NC * NS
PER_W = B // NW
G = 128
K = 13
SC_ROWS = G * K
NSC = PER_W // SC_ROWS
NCH = PER_W // G
MAIN_COLS = 61 * 32 * 512


def _repack_body(tableT_hbm, tail_hbm, packed_hbm, a_v, b_v, sem_in, sem_out):
    wid = lax.axis_index("s") * NC + lax.axis_index("c")
    lane = lax.iota(jnp.int32, 16)
    lane32 = lane * 32
    rots = [(lane + j) % 16 for j in range(16)]

    rot_dt = [r >> 3 for r in rots]
    rot_r = [r & 7 for r in rots]

    def transpose_group(pin, p, nq):
        pvec = jnp.full((16,), pin, jnp.int32)
        pbase = lane32 + p * 16384

        def qbody(q, c2):
            col = q * 16 + lane
            for h in range(2):
                base = pbase + (q * 512 + h * 16)
                for j in range(16):
                    vec = plsc.load_gather(
                        a_v, [pvec, rot_dt[j] + 2 * h, rot_r[j], col])
                    plsc.store_scatter(b_v, [base + rots[j]], vec)
            return c2
        lax.fori_loop(0, nq, qbody, 0, unroll=4)

    def in_copies(i, p):
        grp = i * 32 + wid
        c0 = grp * 512
        return [pltpu.make_async_copy(
            tableT_hbm.at[pl.ds(kk * 8, 8), pl.ds(c0, 512)],
            a_v.at[p, kk], sem_in.at[p]) for kk in range(4)]

    def out_copy(i, p):
        grp = i * 32 + wid
        c0 = grp * 512
        return pltpu.make_async_copy(
            b_v.at[pl.ds(p * 16384, 16384)],
            packed_hbm.at[pl.ds(c0 * 32, 16384)], sem_out.at[p])

    for c in in_copies(0, 0):
        c.start()
    for c in in_copies(1, 1):
        c.start()

    def grp_body(i, carry):
        p3 = lax.rem(i, 3)
        p = lax.rem(i, 2)
        for c in in_copies(i, p3):
            c.wait()

        @pl.when(i + 2 < 61)
        def _prefetch():
            for c in in_copies(i + 2, lax.rem(i + 2, 3)):
                c.start()

        @pl.when(i >= 2)
        def _drain():
            out_copy(i - 2, p).wait()
        transpose_group(p3, p, 32)
        out_copy(i, p).start()
        return carry
    lax.fori_loop(0, 61, grp_body, 0)
    out_copy(59, 1).wait()
    out_copy(60, 0).wait()

    @pl.when(wid < 4)
    def _tail():
        off = wid * 4608
        pltpu.sync_copy(tail_hbm.at[pl.ds(off, 4608)],
                        b_v.at[pl.ds(0, 4608)])
        pltpu.sync_copy(b_v.at[pl.ds(0, 4608)],
                        packed_hbm.at[pl.ds(MAIN_COLS * D + off, 4608)])


def _gather_body(table_hbm, idsT_hbm, valsT_hbm, out_hbm,
                 ids_v, vals_v, rows_v, stage_v, sem_g, sem_o):
    wid = lax.axis_index("s") * NC + lax.axis_index("c")
    b0 = wid * 512
    lane = lax.iota(jnp.int32, 16)
    lane32 = lane * 32
    rots = [(lane + j) % 16 for j in range(16)]
    dparts = [(r >> 3) * 4096 + (r & 7) * 128 for r in rots]

    pltpu.sync_copy(idsT_hbm.at[:, pl.ds(b0, 512)], ids_v)
    pltpu.sync_copy(valsT_hbm.at[:, pl.ds(b0, 512)], vals_v)

    def g_copies(f, p):
        return [pltpu.make_async_copy(
            table_hbm.at[ids_v.at[f, pl.ds(c * 128, 128)]],
            rows_v.at[p, pl.ds(c * 128, 128)], sem_g.at[p])
            for c in range(4)]

    def out_copies(f, p):
        cps = []
        for dt in range(4):
            off = f * 524288 + dt * 131072 + wid * 4096
            cps.append(pltpu.make_async_copy(
                stage_v.at[pl.ds(p * 16384 + dt * 4096, 4096)],
                out_hbm.at[pl.ds(off, 4096)], sem_o.at[p]))
        return cps

    def transpose_scale(f, pin, p):
        pvec = jnp.full((16,), pin, jnp.int32)

        def qbody(q, c2):
            vv = vals_v[f, pl.ds(q * 16, 16)]
            bvec = q * 16 + lane
            sbase = (p * 16384 + (q >> 3) * 1024 + (q & 7) * 16) + lane
            for h in range(2):
                sb2 = sbase + h * 8192
                for j in range(16):
                    dvec = rots[j] + h * 16
                    vec = plsc.load_gather(rows_v, [pvec, bvec, dvec])
                    plsc.store_scatter(stage_v, [sb2 + dparts[j]], vec * vv)
            return c2
        lax.fori_loop(0, 32, qbody, 0, unroll=4)

    for cp in g_copies(0, 0):
        cp.start()
    for cp in g_copies(1, 1):
        cp.start()

    def fbody(f, carry):
        p3 = lax.rem(f, 3)
        p = lax.rem(f, 2)
        for cp in g_copies(f, p3):
            cp.wait()

        @pl.when(f + 2 < FIELDS)
        def _prefetch():
            for cp in g_copies(f + 2, lax.rem(f + 2, 3)):
                cp.start()

        @pl.when(f >= 2)
        def _drain():
            for cp in out_copies(f - 2, p):
                cp.wait()
        transpose_scale(f, p3, p)
        for cp in out_copies(f, p):
            cp.start()
        return carry
    lax.fori_loop(0, FIELDS, fbody, 0)
    for cp in out_copies(FIELDS - 2, 0):
        cp.wait()
    for cp in out_copies(FIELDS - 1, 1):
        cp.wait()


@jax.jit
def kernel(embedding, inputs_id, inputs_value):
    idsT = inputs_id.astype(jnp.int32).T
    valsT = inputs_value.T
    mesh = plsc.VectorSubcoreMesh(core_axis_name="c", subcore_axis_name="s")
    packed = pl.kernel(
        _repack_body,
        mesh=mesh,
        compiler_params=pltpu.CompilerParams(needs_layout_passes=False),
        out_type=jax.ShapeDtypeStruct((VOCAB * D,), jnp.float32),
        scratch_types=[
            pltpu.VMEM((3, 4, 8, 512), jnp.float32),
            pltpu.VMEM((32768,), jnp.float32),
            pltpu.SemaphoreType.DMA((3,)),
            pltpu.SemaphoreType.DMA((2,)),
        ],
    )(embedding.T, embedding[MAIN_COLS:, :].reshape(-1))
    table_lin = packed.reshape(VOCAB, D)
    out_flat = pl.kernel(
        _gather_body,
        mesh=mesh,
        compiler_params=pltpu.CompilerParams(
            use_tc_tiling_on_sc=False, needs_layout_passes=False),
        out_type=jax.ShapeDtypeStruct((B * D,), jnp.float32),
        scratch_types=[
            pltpu.VMEM((FIELDS, 512), jnp.int32),
            pltpu.VMEM((FIELDS, 512), jnp.float32),
            pltpu.VMEM((3, 512, D), jnp.float32),
            pltpu.VMEM((32768,), jnp.float32),
            pltpu.SemaphoreType.DMA((3,)),
            pltpu.SemaphoreType.DMA((2,)),
        ],
    )(table_lin, idsT, valsT)
    out5 = out_flat.reshape(FIELDS, 4, 128, 8, 128)
    return out5.transpose(2, 4, 0, 1, 3).reshape(BATCH, FIELDS, D)


# speedup vs baseline: 1.0559x; 1.0559x over previous
import jax
import jax.numpy as jnp
from jax import lax
from jax.experimental import pallas as pl
from jax.experimental.pallas import tpu as pltpu
from jax.experimental.pallas import tpu_sc as plsc

VOCAB = 1000000
D = 32
BATCH = 16384
FIELDS = 26
B = BATCH * FIELDS

NC = 2
NS = 16
NW = NC * NS
PER_W = B // NW
G = 128
K = 13
SC_ROWS = G * K
NSC = PER_W // SC_ROWS
NCH = PER_W // G
MAIN_COLS = 61 * 32 * 512


def _repack_body(tableT_hbm, tail_hbm, packed_hbm, a_v, b_v, sem_in, sem_out):
    wid = lax.axis_index("s") * NC + lax.axis_index("c")
    lane = lax.iota(jnp.int32, 16)
    lane32 = lane * 32
    rots = [(lane + j) % 16 for j in range(16)]

    def transpose_group(pin, p, nq):
        pvec = jnp.full((16,), pin, jnp.int32)
        pbase = lane32 + p * 16384

        def qbody(q, c2):
            k = jnp.full((16,), q // 8, jnp.int32)
            col = (q % 8) * 16 + lane
            for h in range(2):
                base = pbase + (q * 512 + h * 16)
                for j in range(16):
                    row = rots[j] + (h * 16)
                    vec = plsc.load_gather(a_v, [pvec, k, row, col])
                    plsc.store_scatter(b_v, [base + rots[j]], vec)
            return c2
        lax.fori_loop(0, nq, qbody, 0)

    def in_copies(i, p):
        grp = i * 32 + wid
        c0 = grp * 512
        return [pltpu.make_async_copy(
            tableT_hbm.at[:, pl.ds(c0 + kk * 128, 128)],
            a_v.at[p, kk], sem_in.at[p]) for kk in range(4)]

    def out_copy(i, p):
        grp = i * 32 + wid
        c0 = grp * 512
        return pltpu.make_async_copy(
            b_v.at[pl.ds(p * 16384, 16384)],
            packed_hbm.at[pl.ds(c0 * 32, 16384)], sem_out.at[p])

    for c in in_copies(0, 0):
        c.start()

    def grp_body(i, carry):
        p = lax.rem(i, 2)
        for c in in_copies(i, p):
            c.wait()

        @pl.when(i + 1 < 61)
        def _prefetch():
            for c in in_copies(i + 1, 1 - p):
                c.start()

        @pl.when(i >= 2)
        def _drain():
            out_copy(i - 2, p).wait()
        transpose_group(p, p, 32)
        out_copy(i, p).start()
        return carry
    lax.fori_loop(0, 61, grp_body, 0)
    out_copy(59, 1).wait()
    out_copy(60, 0).wait()

    @pl.when(wid < 4)
    def _tail():
        off = wid * 4608
        pltpu.sync_copy(tail_hbm.at[pl.ds(off, 4608)],
                        b_v.at[pl.ds(0, 4608)])
        pltpu.sync_copy(b_v.at[pl.ds(0, 4608)],
                        packed_hbm.at[pl.ds(MAIN_COLS * D + off, 4608)])


def _gather_body(table_hbm, idsT_hbm, valsT_hbm, out_hbm,
                 ids_v, vals_v, rows_v, stage_v, sem_g, sem_o):
    wid = lax.axis_index("s") * NC + lax.axis_index("c")
    b0 = wid * 512
    lane = lax.iota(jnp.int32, 16)
    lane32 = lane * 32
    rots = [(lane + j) % 16 for j in range(16)]
    dparts = [(r >> 3) * 4096 + (r & 7) * 128 for r in rots]

    pltpu.sync_copy(idsT_hbm.at[:, pl.ds(b0, 512)], ids_v)
    pltpu.sync_copy(valsT_hbm.at[:, pl.ds(b0, 512)], vals_v)

    def g_copies(f, p):
        return [pltpu.make_async_copy(
            table_hbm.at[ids_v.at[f, pl.ds(c * 128, 128)]],
            rows_v.at[p, pl.ds(c * 128, 128)], sem_g.at[p])
            for c in range(4)]

    def out_copies(f, p):
        cps = []
        for dt in range(4):
            off = f * 524288 + dt * 131072 + wid * 4096
            cps.append(pltpu.make_async_copy(
                stage_v.at[pl.ds(p * 16384 + dt * 4096, 4096)],
                out_hbm.at[pl.ds(off, 4096)], sem_o.at[p]))
        return cps

    def transpose_scale(f, pin, p):
        pvec = jnp.full((16,), pin, jnp.int32)

        def qbody(q, c2):
            vv = vals_v[f, pl.ds(q * 16, 16)]
            bvec = q * 16 + lane
            sbase = (p * 16384 + (q >> 3) * 1024 + (q & 7) * 16) + lane
            for h in range(2):
                sb2 = sbase + h * 8192
                for j in range(16):
                    dvec = rots[j] + h * 16
                    vec = plsc.load_gather(rows_v, [pvec, bvec, dvec])
                    plsc.store_scatter(stage_v, [sb2 + dparts[j]], vec * vv)
            return c2
        lax.fori_loop(0, 32, qbody, 0)

    for cp in g_copies(0, 0):
        cp.start()

    def fbody(f, carry):
        p = lax.rem(f, 2)
        for cp in g_copies(f, p):
            cp.wait()

        @pl.when(f + 1 < FIELDS)
        def _prefetch():
            for cp in g_copies(f + 1, 1 - p):
                cp.start()

        @pl.when(f >= 2)
        def _drain():
            for cp in out_copies(f - 2, p):
                cp.wait()
        transpose_scale(f, p, p)
        for cp in out_copies(f, p):
            cp.start()
        return carry
    lax.fori_loop(0, FIELDS, fbody, 0)
    for cp in out_copies(FIELDS - 2, 0):
        cp.wait()
    for cp in out_copies(FIELDS - 1, 1):
        cp.wait()


@jax.jit
def kernel(embedding, inputs_id, inputs_value):
    idsT = inputs_id.astype(jnp.int32).T
    valsT = inputs_value.T
    mesh = plsc.VectorSubcoreMesh(core_axis_name="c", subcore_axis_name="s")
    packed = pl.kernel(
        _repack_body,
        mesh=mesh,
        compiler_params=pltpu.CompilerParams(needs_layout_passes=False),
        out_type=jax.ShapeDtypeStruct((VOCAB * D,), jnp.float32),
        scratch_types=[
            pltpu.VMEM((2, 4, 32, 128), jnp.float32),
            pltpu.VMEM((32768,), jnp.float32),
            pltpu.SemaphoreType.DMA((2,)),
            pltpu.SemaphoreType.DMA((2,)),
        ],
    )(embedding.T, embedding[MAIN_COLS:, :].reshape(-1))
    table_lin = packed.reshape(VOCAB, D)
    out_flat = pl.kernel(
        _gather_body,
        mesh=mesh,
        compiler_params=pltpu.CompilerParams(
            use_tc_tiling_on_sc=False, needs_layout_passes=False),
        out_type=jax.ShapeDtypeStruct((B * D,), jnp.float32),
        scratch_types=[
            pltpu.VMEM((FIELDS, 512), jnp.int32),
            pltpu.VMEM((FIELDS, 512), jnp.float32),
            pltpu.VMEM((2, 512, D), jnp.float32),
            pltpu.VMEM((32768,), jnp.float32),
            pltpu.SemaphoreType.DMA((2,)),
            pltpu.SemaphoreType.DMA((2,)),
        ],
    )(table_lin, idsT, valsT)
    out5 = out_flat.reshape(FIELDS, 4, 128, 8, 128)
    return out5.transpose(2, 4, 0, 1, 3).reshape(BATCH, FIELDS, D)
